# baseline (speedup 1.0000x reference)
import functools
import math

import jax
import jax.numpy as jnp
from jax import lax
from jax.experimental import pallas as pl
from jax.experimental.pallas import tpu as pltpu
from jax.experimental.pallas import tpu_sc as plsc

K = 10
B, N, PD = 64, 1024, 96
BI = 256
KP = 16
RS = 512


def _gelu(x):
    return 0.5 * x * (1.0 + lax.erf(x * (1.0 / math.sqrt(2.0))))


def _make_sc_topk(nb):
    nw = 32
    ntasks = nb * (N // RS)
    tpw = ntasks // nw

    mesh = plsc.VectorSubcoreMesh(core_axis_name="c", subcore_axis_name="s",
                                  num_cores=2)

    @functools.partial(
        pl.kernel, mesh=mesh,
        out_type=jax.ShapeDtypeStruct((nb * N * KP,), jnp.int32),
        scratch_types=[
            pltpu.VMEM((N,), jnp.float32),
            pltpu.VMEM((N,), jnp.float32),
            pltpu.VMEM((N,), jnp.float32),
            pltpu.VMEM((RS * 16,), jnp.float32),
            pltpu.VMEM((RS * 16,), jnp.float32),
            pltpu.VMEM((RS * 16,), jnp.float32),
            pltpu.VMEM((RS * KP,), jnp.int32),
        ],
        compiler_params=pltpu.CompilerParams(needs_layout_passes=False),
    )
    def sc_topk(xgb_h, ygb_h, rb_h, xrb_h, yrb_h, ra_h, out_h,
                xg_v, yg_v, rb_v, xr_v, yr_v, ra_v, idx_v):
        wid = lax.axis_index("s") * 2 + lax.axis_index("c")
        lane = lax.iota(jnp.int32, 16)
        inf16 = jnp.full((16,), jnp.inf, dtype=jnp.float32)

        for q in range(tpw):
            task = wid * tpw + q
            b = task // (N // RS)
            half = task % (N // RS)
            gbase = b * N
            rbase = b * N + half * RS
            pltpu.sync_copy(xgb_h.at[pl.ds(gbase, N)], xg_v)
            pltpu.sync_copy(ygb_h.at[pl.ds(gbase, N)], yg_v)
            pltpu.sync_copy(rb_h.at[pl.ds(gbase, N)], rb_v)
            pltpu.sync_copy(xrb_h.at[pl.ds(rbase * 16, RS * 16)], xr_v)
            pltpu.sync_copy(yrb_h.at[pl.ds(rbase * 16, RS * 16)], yr_v)
            pltpu.sync_copy(ra_h.at[pl.ds(rbase * 16, RS * 16)], ra_v)

            nrl = 8

            def row_body(i, _):
                xis, yis, rais = [], [], []
                for u in range(nrl):
                    r0 = (i + u * (RS // nrl)) * 16
                    xis.append(xr_v[pl.ds(r0, 16)])
                    yis.append(yr_v[pl.ds(r0, 16)])
                    rais.append(ra_v[pl.ds(r0, 16)])

                def chunk_body(c, carry):
                    off = c * 16
                    xc = xg_v[pl.ds(off, 16)]
                    yc = yg_v[pl.ds(off, 16)]
                    rc = rb_v[pl.ds(off, 16)]
                    jc = lane + off
                    out = []
                    for u in range(nrl):
                        sk, sv = carry[2 * u], carry[2 * u + 1]
                        m = xis[u] * xc + yis[u] * yc
                        t = rais[u] - 2.0 * m
                        dc = (t + rc) + 1e-05
                        dk, jv = plsc.sort_key_val(dc, jc, descending=True)
                        cmp = dk < sk
                        nk = jnp.where(cmp, dk, sk)
                        nv = jnp.where(cmp, jv, sv)
                        rk, rv = plsc.sort_key_val(nk, nv, descending=False)
                        out += [rk, rv]
                    return tuple(out)

                init = (inf16, jnp.zeros((16,), jnp.int32)) * nrl
                res = lax.fori_loop(0, N // 16, chunk_body, init)
                for u in range(nrl):
                    r0 = (i + u * (RS // nrl)) * KP
                    idx_v[pl.ds(r0, KP)] = res[2 * u + 1]
                return 0

            lax.fori_loop(0, RS // nrl, row_body, 0)
            pltpu.sync_copy(idx_v, out_h.at[pl.ds(rbase * KP, RS * KP)])

    return sc_topk


def _tc_mlp_kernel(idx_ref, fr_ref, fg_ref, w1a_ref, w1d_ref, b1_ref,
                   w2_ref, b2_ref, out_ref):
    fg = fg_ref[0]
    fr = fr_ref[0]
    c = jnp.dot(fr, w1d_ref[...], preferred_element_type=jnp.float32)
    c = c + b1_ref[...]

    idx = idx_ref[0]
    iota = lax.broadcasted_iota(jnp.int32, (BI, N), 1)
    fgb = fg.astype(jnp.bfloat16)
    acc = jnp.full((BI, PD), -jnp.inf, dtype=jnp.float32)
    for r in range(K):
        sel = iota == idx[:, r:r + 1]
        oh = sel.astype(jnp.bfloat16)
        fk = jnp.dot(oh, fgb, preferred_element_type=jnp.float32)
        h = jnp.dot(fk, w1a_ref[...], preferred_element_type=jnp.float32) + c
        h = _gelu(h)
        h2 = jnp.dot(h, w2_ref[...], preferred_element_type=jnp.float32)
        h2 = _gelu(h2 + b2_ref[...])
        acc = jnp.maximum(acc, h2)
    out_ref[0] = acc


def _tc_mlp(nb, idx, features_reco, features_gen, w1a, w1d, b1r, W2, b2r):
    grid = (nb, N // BI)
    return pl.pallas_call(
        _tc_mlp_kernel,
        grid=grid,
        in_specs=[
            pl.BlockSpec((1, BI, KP), lambda b, i: (b, i, 0)),
            pl.BlockSpec((1, BI, PD), lambda b, i: (b, i, 0)),
            pl.BlockSpec((1, N, PD), lambda b, i: (b, 0, 0)),
            pl.BlockSpec((PD, 4 * PD), lambda b, i: (0, 0)),
            pl.BlockSpec((PD, 4 * PD), lambda b, i: (0, 0)),
            pl.BlockSpec((1, 4 * PD), lambda b, i: (0, 0)),
            pl.BlockSpec((4 * PD, PD), lambda b, i: (0, 0)),
            pl.BlockSpec((1, PD), lambda b, i: (0, 0)),
        ],
        out_specs=pl.BlockSpec((1, BI, PD), lambda b, i: (b, i, 0)),
        out_shape=jax.ShapeDtypeStruct((nb, N, PD), jnp.float32),
        compiler_params=pltpu.CompilerParams(
            dimension_semantics=("parallel", "parallel"),
        ),
    )(idx, features_reco, features_gen, w1a, w1d, b1r, W2, b2r)


@jax.jit
def kernel(points_reco, points_gen, features_reco, features_gen, W1, b1, W2, b2):
    xr = points_reco[..., 0]
    yr = points_reco[..., 1]
    xg = points_gen[..., 0]
    yg = points_gen[..., 1]
    xrh, yrh, xgh, ygh = lax.optimization_barrier(
        (xr.astype(jnp.bfloat16), yr.astype(jnp.bfloat16),
         xg.astype(jnp.bfloat16), yg.astype(jnp.bfloat16)))
    xrb = xrh.astype(jnp.float32)
    yrb = yrh.astype(jnp.float32)
    xgb = xgh.astype(jnp.float32)
    ygb = ygh.astype(jnp.float32)
    ra = xr * xr + yr * yr
    rb = xg * xg + yg * yg
    w1a = W1[:PD]
    w1d = W1[PD:] - W1[:PD]
    b1r = b1.reshape(1, 4 * PD)
    b2r = b2.reshape(1, PD)

    nh = 2
    nb = B // nh
    sc_topk = _make_sc_topk(nb)
    sl = [slice(h * nb, (h + 1) * nb) for h in range(nh)]
    xrb16 = jnp.repeat(xrb.reshape(-1), 16)
    yrb16 = jnp.repeat(yrb.reshape(-1), 16)
    ra16 = jnp.repeat(ra.reshape(-1), 16)
    sz = nb * N * 16
    idxs = [
        sc_topk(xgb[sl[h]].reshape(-1), ygb[sl[h]].reshape(-1),
                rb[sl[h]].reshape(-1),
                lax.dynamic_slice(xrb16, (h * sz,), (sz,)),
                lax.dynamic_slice(yrb16, (h * sz,), (sz,)),
                lax.dynamic_slice(ra16, (h * sz,), (sz,)))
        for h in range(nh)
    ]
    outs = [
        _tc_mlp(nb, idxs[h].reshape(nb, N, KP), features_reco[sl[h]],
                features_gen[sl[h]], w1a, w1d, b1r, W2, b2r)
        for h in range(nh)
    ]
    return jnp.concatenate(outs, axis=0)


# speedup vs baseline: 1.3071x; 1.3071x over previous
import functools
import math

import jax
import jax.numpy as jnp
from jax import lax
from jax.experimental import pallas as pl
from jax.experimental.pallas import tpu as pltpu
from jax.experimental.pallas import tpu_sc as plsc

K = 10
B, N, PD = 64, 1024, 96
BI = 256
KP = 16
RS = 512


def _gelu(x):
    return 0.5 * x * (1.0 + lax.erf(x * (1.0 / math.sqrt(2.0))))


def _make_sc_topk(nb):
    nw = 32
    ntasks = nb * (N // RS)
    tpw = ntasks // nw

    mesh = plsc.VectorSubcoreMesh(core_axis_name="c", subcore_axis_name="s",
                                  num_cores=2)

    @functools.partial(
        pl.kernel, mesh=mesh,
        out_type=jax.ShapeDtypeStruct((nb * N * KP,), jnp.int32),
        scratch_types=[
            pltpu.VMEM((N,), jnp.float32),
            pltpu.VMEM((N,), jnp.float32),
            pltpu.VMEM((N,), jnp.float32),
            pltpu.VMEM((RS * 16,), jnp.float32),
            pltpu.VMEM((RS * 16,), jnp.float32),
            pltpu.VMEM((RS * 16,), jnp.float32),
            pltpu.VMEM((RS * KP,), jnp.int32),
        ],
        compiler_params=pltpu.CompilerParams(needs_layout_passes=False),
    )
    def sc_topk(xgb_h, ygb_h, rb_h, xrb_h, yrb_h, ra_h, out_h,
                xg_v, yg_v, rb_v, xr_v, yr_v, ra_v, idx_v):
        wid = lax.axis_index("s") * 2 + lax.axis_index("c")
        lane = lax.iota(jnp.int32, 16)
        inf16 = jnp.full((16,), jnp.inf, dtype=jnp.float32)

        for q in range(tpw):
            task = wid * tpw + q
            b = task // (N // RS)
            half = task % (N // RS)
            gbase = b * N
            rbase = b * N + half * RS
            pltpu.sync_copy(xgb_h.at[pl.ds(gbase, N)], xg_v)
            pltpu.sync_copy(ygb_h.at[pl.ds(gbase, N)], yg_v)
            pltpu.sync_copy(rb_h.at[pl.ds(gbase, N)], rb_v)
            pltpu.sync_copy(xrb_h.at[pl.ds(rbase * 16, RS * 16)], xr_v)
            pltpu.sync_copy(yrb_h.at[pl.ds(rbase * 16, RS * 16)], yr_v)
            pltpu.sync_copy(ra_h.at[pl.ds(rbase * 16, RS * 16)], ra_v)

            nrl = 8

            def row_body(i, _):
                xis, yis, rais = [], [], []
                for u in range(nrl):
                    r0 = (i + u * (RS // nrl)) * 16
                    xis.append(xr_v[pl.ds(r0, 16)])
                    yis.append(yr_v[pl.ds(r0, 16)])
                    rais.append(ra_v[pl.ds(r0, 16)])

                def chunk_body(c, carry):
                    off = c * 16
                    xc = xg_v[pl.ds(off, 16)]
                    yc = yg_v[pl.ds(off, 16)]
                    rc = rb_v[pl.ds(off, 16)]
                    jc = lane + off
                    out = []
                    for u in range(nrl):
                        sk, sv = carry[2 * u], carry[2 * u + 1]
                        m = xis[u] * xc + yis[u] * yc
                        t = rais[u] - 2.0 * m
                        dc = (t + rc) + 1e-05
                        dk, jv = plsc.sort_key_val(dc, jc, descending=True)
                        cmp = dk < sk
                        nk = jnp.where(cmp, dk, sk)
                        nv = jnp.where(cmp, jv, sv)
                        rk, rv = plsc.sort_key_val(nk, nv, descending=False)
                        out += [rk, rv]
                    return tuple(out)

                init = (inf16, jnp.zeros((16,), jnp.int32)) * nrl
                res = lax.fori_loop(0, N // 16, chunk_body, init)
                for u in range(nrl):
                    r0 = (i + u * (RS // nrl)) * KP
                    idx_v[pl.ds(r0, KP)] = res[2 * u + 1]
                return 0

            lax.fori_loop(0, RS // nrl, row_body, 0)
            pltpu.sync_copy(idx_v, out_h.at[pl.ds(rbase * KP, RS * KP)])

    return sc_topk


def _tc_mlp_kernel(idx_ref, fr_ref, fg_ref, w1a_ref, w1d_ref, b1_ref,
                   w2_ref, b2_ref, out_ref):
    fg = fg_ref[0]
    fr = fr_ref[0]
    c = jnp.dot(fr, w1d_ref[...], preferred_element_type=jnp.float32)
    c = c + b1_ref[...]

    idx = idx_ref[0]
    iota = lax.broadcasted_iota(jnp.int32, (BI, N), 1)
    fgb = fg.astype(jnp.bfloat16)
    fks = []
    for r in range(K):
        sel = iota == idx[:, r:r + 1]
        oh = sel.astype(jnp.bfloat16)
        fks.append(jnp.dot(oh, fgb, preferred_element_type=jnp.float32))
    fk = jnp.concatenate(fks, axis=0)
    h = jnp.dot(fk, w1a_ref[...], preferred_element_type=jnp.float32)
    h = _gelu(h + jnp.concatenate([c] * K, axis=0))
    h2 = jnp.dot(h, w2_ref[...], preferred_element_type=jnp.float32)
    h2 = _gelu(h2 + b2_ref[...])
    acc = h2[:BI]
    for r in range(1, K):
        acc = jnp.maximum(acc, h2[r * BI:(r + 1) * BI])
    out_ref[0] = acc


def _tc_mlp(nb, idx, features_reco, features_gen, w1a, w1d, b1r, W2, b2r):
    grid = (nb, N // BI)
    return pl.pallas_call(
        _tc_mlp_kernel,
        grid=grid,
        in_specs=[
            pl.BlockSpec((1, BI, KP), lambda b, i: (b, i, 0)),
            pl.BlockSpec((1, BI, PD), lambda b, i: (b, i, 0)),
            pl.BlockSpec((1, N, PD), lambda b, i: (b, 0, 0)),
            pl.BlockSpec((PD, 4 * PD), lambda b, i: (0, 0)),
            pl.BlockSpec((PD, 4 * PD), lambda b, i: (0, 0)),
            pl.BlockSpec((1, 4 * PD), lambda b, i: (0, 0)),
            pl.BlockSpec((4 * PD, PD), lambda b, i: (0, 0)),
            pl.BlockSpec((1, PD), lambda b, i: (0, 0)),
        ],
        out_specs=pl.BlockSpec((1, BI, PD), lambda b, i: (b, i, 0)),
        out_shape=jax.ShapeDtypeStruct((nb, N, PD), jnp.float32),
        compiler_params=pltpu.CompilerParams(
            dimension_semantics=("parallel", "parallel"),
        ),
    )(idx, features_reco, features_gen, w1a, w1d, b1r, W2, b2r)


@jax.jit
def kernel(points_reco, points_gen, features_reco, features_gen, W1, b1, W2, b2):
    xr = points_reco[..., 0]
    yr = points_reco[..., 1]
    xg = points_gen[..., 0]
    yg = points_gen[..., 1]
    xrh, yrh, xgh, ygh = lax.optimization_barrier(
        (xr.astype(jnp.bfloat16), yr.astype(jnp.bfloat16),
         xg.astype(jnp.bfloat16), yg.astype(jnp.bfloat16)))
    xrb = xrh.astype(jnp.float32)
    yrb = yrh.astype(jnp.float32)
    xgb = xgh.astype(jnp.float32)
    ygb = ygh.astype(jnp.float32)
    ra = xr * xr + yr * yr
    rb = xg * xg + yg * yg
    w1a = W1[:PD]
    w1d = W1[PD:] - W1[:PD]
    b1r = b1.reshape(1, 4 * PD)
    b2r = b2.reshape(1, PD)

    nh = 4
    nb = B // nh
    sc_topk = _make_sc_topk(nb)
    sl = [slice(h * nb, (h + 1) * nb) for h in range(nh)]
    xrb16 = jnp.repeat(xrb.reshape(-1), 16)
    yrb16 = jnp.repeat(yrb.reshape(-1), 16)
    ra16 = jnp.repeat(ra.reshape(-1), 16)
    sz = nb * N * 16
    idxs = [
        sc_topk(xgb[sl[h]].reshape(-1), ygb[sl[h]].reshape(-1),
                rb[sl[h]].reshape(-1),
                lax.dynamic_slice(xrb16, (h * sz,), (sz,)),
                lax.dynamic_slice(yrb16, (h * sz,), (sz,)),
                lax.dynamic_slice(ra16, (h * sz,), (sz,)))
        for h in range(nh)
    ]
    outs = [
        _tc_mlp(nb, idxs[h].reshape(nb, N, KP), features_reco[sl[h]],
                features_gen[sl[h]], w1a, w1d, b1r, W2, b2r)
        for h in range(nh)
    ]
    return jnp.concatenate(outs, axis=0)
